# Initial kernel scaffold; baseline (speedup 1.0000x reference)
import jax
import jax.numpy as jnp
from jax.experimental import pallas as pl


def kernel(x, edge_index, edge_attr, e1_w1, e1_b1, e1_w2, e1_b2, n1a_w1, n1a_b1, n1a_w2, n1a_b2, n1b_w1, n1b_b1, n1b_w2, n1b_b2, e2_w1, e2_b1, e2_w2, e2_b2, n2a_w1, n2a_b1, n2a_w2, n2a_b2, n2b_w1, n2b_b1, n2b_w2, n2b_b2):
    raise NotImplementedError("write your pallas kernel here")


# speedup vs baseline: 2.4895x; 2.4895x over previous
import functools

import jax
import jax.numpy as jnp
from jax import lax
from jax.experimental import pallas as pl
from jax.experimental.pallas import tpu as pltpu
from jax.experimental.pallas import tpu_sc as plsc

_NC = 2
_NS = 16
_NW = _NC * _NS
_CHUNK = 80


def _sc_mesh():
    return plsc.VectorSubcoreMesh(
        core_axis_name="c", subcore_axis_name="s",
        num_cores=_NC, num_subcores=_NS)


def _gather_two(table, row, col):
    n, hd = table.shape
    e = row.shape[0]
    ept = e // _NW
    nch = ept // _CHUNK
    assert ept * _NW == e and nch * _CHUNK == ept

    @functools.partial(
        pl.kernel,
        out_type=[jax.ShapeDtypeStruct((e, hd), table.dtype),
                  jax.ShapeDtypeStruct((e, hd), table.dtype)],
        mesh=_sc_mesh(),
        scratch_types=[pltpu.VMEM((_CHUNK,), jnp.int32),
                       pltpu.VMEM((_CHUNK,), jnp.int32),
                       pltpu.VMEM((_CHUNK, hd), table.dtype),
                       pltpu.VMEM((_CHUNK, hd), table.dtype),
                       pltpu.SemaphoreType.DMA,
                       pltpu.SemaphoreType.DMA],
    )
    def gk(tab_hbm, row_hbm, col_hbm, outr_hbm, outc_hbm,
           idxr_v, idxc_v, bufr_v, bufc_v, semr, semc):
        wid = lax.axis_index("s") * _NC + lax.axis_index("c")
        base = wid * ept

        def body(i, carry):
            off = base + i * _CHUNK
            pltpu.sync_copy(row_hbm.at[pl.ds(off, _CHUNK)], idxr_v)
            pltpu.sync_copy(col_hbm.at[pl.ds(off, _CHUNK)], idxc_v)
            cr = pltpu.async_copy(tab_hbm.at[idxr_v], bufr_v, semr)
            cc = pltpu.async_copy(tab_hbm.at[idxc_v], bufc_v, semc)
            cr.wait()
            cc.wait()
            pltpu.sync_copy(bufr_v, outr_hbm.at[pl.ds(off, _CHUNK), :])
            pltpu.sync_copy(bufc_v, outc_hbm.at[pl.ds(off, _CHUNK), :])
            return carry

        lax.fori_loop(0, nch, body, 0)

    return gk(table, row, col)


def _acc_pad(n):
    npt = -(-n // (_NS * _CHUNK)) * _CHUNK
    return npt, npt * _NS


def _scatter_sum(vals, row, n, token):
    e, hd = vals.shape
    ept = e // _NW
    nch = ept // _CHUNK
    npt, np_ = _acc_pad(n)
    assert ept * _NW == e and nch * _CHUNK == ept

    zeros_h = jnp.zeros((np_, hd), vals.dtype)

    @functools.partial(
        pl.kernel,
        out_type=[jax.ShapeDtypeStruct((_NC, np_, hd), vals.dtype)],
        mesh=_sc_mesh(),
        scratch_types=[pltpu.VMEM((_CHUNK,), jnp.int32),
                       pltpu.VMEM((_CHUNK, hd), vals.dtype),
                       pltpu.VMEM_SHARED((np_, hd), vals.dtype)],
    )
    def sk(vals_hbm, row_hbm, zh_hbm, tok_hbm, sums_hbm, idx_v, vals_v,
           acc_sh):
        cid = lax.axis_index("c")
        sid = lax.axis_index("s")
        base = (sid * _NC + cid) * ept
        nbase = sid * npt
        nzch = npt // _CHUNK
        pltpu.sync_copy(zh_hbm.at[pl.ds(0, _CHUNK), :], vals_v)

        def zbody(i, carry):
            pltpu.sync_copy(vals_v, acc_sh.at[pl.ds(nbase + i * _CHUNK, _CHUNK), :])
            return carry

        lax.fori_loop(0, nzch, zbody, 0)
        plsc.subcore_barrier()

        def body(i, carry):
            off = base + i * _CHUNK
            pltpu.sync_copy(row_hbm.at[pl.ds(off, _CHUNK)], idx_v)
            pltpu.sync_copy(vals_hbm.at[pl.ds(off, _CHUNK), :], vals_v)
            pltpu.sync_copy(vals_v, acc_sh.at[idx_v], add=True)
            return carry

        lax.fori_loop(0, nch, body, 0)
        plsc.subcore_barrier()

        def wbody(i, carry):
            zoff = nbase + i * _CHUNK
            pltpu.sync_copy(acc_sh.at[pl.ds(zoff, _CHUNK), :], vals_v)
            pltpu.sync_copy(vals_v, sums_hbm.at[cid, pl.ds(zoff, _CHUNK), :])
            return carry

        lax.fori_loop(0, nzch, wbody, 0)

    return sk(vals, row, zeros_h, token)[0]


def _scatter_ones(row, n, hd, dtype, token):
    e = row.shape[0]
    ept = e // _NW
    nch = ept // _CHUNK
    npt, np_ = _acc_pad(n)

    zeros_h = jnp.zeros((np_, hd), dtype)
    ones = jnp.ones((_CHUNK, hd), dtype)

    @functools.partial(
        pl.kernel,
        out_type=[jax.ShapeDtypeStruct((_NC, np_, hd), dtype)],
        mesh=_sc_mesh(),
        scratch_types=[pltpu.VMEM((_CHUNK,), jnp.int32),
                       pltpu.VMEM((_CHUNK, hd), dtype),
                       pltpu.VMEM((_CHUNK, hd), dtype),
                       pltpu.VMEM_SHARED((np_, hd), dtype)],
    )
    def ck(row_hbm, zh_hbm, ones_hbm, tok_hbm, cnt_hbm, idx_v, ones_v,
           stage_v, acc_sh):
        cid = lax.axis_index("c")
        sid = lax.axis_index("s")
        base = (sid * _NC + cid) * ept
        nbase = sid * npt
        nzch = npt // _CHUNK
        pltpu.sync_copy(zh_hbm.at[pl.ds(0, _CHUNK), :], stage_v)

        def zbody(i, carry):
            pltpu.sync_copy(stage_v, acc_sh.at[pl.ds(nbase + i * _CHUNK, _CHUNK), :])
            return carry

        lax.fori_loop(0, nzch, zbody, 0)
        pltpu.sync_copy(ones_hbm, ones_v)
        plsc.subcore_barrier()

        def body(i, carry):
            off = base + i * _CHUNK
            pltpu.sync_copy(row_hbm.at[pl.ds(off, _CHUNK)], idx_v)
            pltpu.sync_copy(ones_v, acc_sh.at[idx_v], add=True)
            return carry

        lax.fori_loop(0, nch, body, 0)
        plsc.subcore_barrier()

        def wbody(i, carry):
            zoff = nbase + i * _CHUNK
            pltpu.sync_copy(acc_sh.at[pl.ds(zoff, _CHUNK), :], stage_v)
            pltpu.sync_copy(stage_v, cnt_hbm.at[cid, pl.ds(zoff, _CHUNK), :])
            return carry

        lax.fori_loop(0, nzch, wbody, 0)

    return ck(row, zeros_h, ones, token)[0]


def _wspec(w):
    nd = w.ndim
    return pl.BlockSpec(w.shape, lambda i, _nd=nd: (0,) * _nd)


_EB = 3200


def _edge1(gr, gc, eattr, w1a, w1b, w1c, b1, w2, b2,
           nw1x, nw1e, nb1, nw2, nb2):
    e, hd = gr.shape
    de = eattr.shape[1]
    grid = e // _EB
    assert grid * _EB == e

    def body(gr_ref, gc_ref, at_ref, w1a_ref, w1b_ref, w1c_ref, b1_ref,
             w2_ref, b2_ref, nw1x_ref, nw1e_ref, nb1_ref, nw2_ref, nb2_ref,
             ea_ref, h_ref):
        grb = gr_ref[...]
        gcb = gc_ref[...]
        t = (grb @ w1a_ref[...] + gcb @ w1b_ref[...]
             + at_ref[...] @ w1c_ref[...] + b1_ref[...])
        ea = jnp.maximum(t, 0.0) @ w2_ref[...] + b2_ref[...]
        ea_ref[...] = ea
        u = gcb @ nw1x_ref[...] + ea @ nw1e_ref[...] + nb1_ref[...]
        h_ref[...] = jnp.maximum(u, 0.0) @ nw2_ref[...] + nb2_ref[...]

    espec = pl.BlockSpec((_EB, hd), lambda i: (i, 0))
    aspec = pl.BlockSpec((_EB, de), lambda i: (i, 0))
    ws = [w1a, w1b, w1c, b1, w2, b2, nw1x, nw1e, nb1, nw2, nb2]
    return pl.pallas_call(
        body,
        grid=(grid,),
        in_specs=[espec, espec, aspec] + [_wspec(w) for w in ws],
        out_specs=[espec, espec],
        out_shape=[jax.ShapeDtypeStruct((e, hd), jnp.float32)] * 2,
    )(gr, gc, eattr, *ws)


def _edge2(gr, gc, ea, w1a, w1b, w1c, b1, w2, b2,
           nw1x, nw1e, nb1, nw2, nb2):
    e, hd = gr.shape
    grid = e // _EB
    assert grid * _EB == e

    def body(gr_ref, gc_ref, ea_ref, w1a_ref, w1b_ref, w1c_ref, b1_ref,
             w2_ref, b2_ref, nw1x_ref, nw1e_ref, nb1_ref, nw2_ref, nb2_ref,
             h_ref):
        grb = gr_ref[...]
        gcb = gc_ref[...]
        t = (grb @ w1a_ref[...] + gcb @ w1b_ref[...]
             + ea_ref[...] @ w1c_ref[...] + b1_ref[...])
        ea2 = jnp.maximum(t, 0.0) @ w2_ref[...] + b2_ref[...]
        u = gcb @ nw1x_ref[...] + ea2 @ nw1e_ref[...] + nb1_ref[...]
        h_ref[...] = jnp.maximum(u, 0.0) @ nw2_ref[...] + nb2_ref[...]

    espec = pl.BlockSpec((_EB, hd), lambda i: (i, 0))
    ws = [w1a, w1b, w1c, b1, w2, b2, nw1x, nw1e, nb1, nw2, nb2]
    return pl.pallas_call(
        body,
        grid=(grid,),
        in_specs=[espec, espec, espec] + [_wspec(w) for w in ws],
        out_specs=espec,
        out_shape=jax.ShapeDtypeStruct((e, hd), jnp.float32),
    )(gr, gc, ea, *ws)


_NB = 2000


def _node(x_in, sums, cnt, w1x, w1a, b1, w2, b2, relu_out):
    n, hd = x_in.shape
    grid = n // _NB
    assert grid * _NB == n

    def body(x_ref, s_ref, c_ref, w1x_ref, w1a_ref, b1_ref, w2_ref, b2_ref,
             o_ref):
        s = s_ref[0] + s_ref[1]
        c = c_ref[0] + c_ref[1]
        agg = s / jnp.maximum(c[:, 0:1], 1.0)
        t = x_ref[...] @ w1x_ref[...] + agg @ w1a_ref[...] + b1_ref[...]
        o = jnp.maximum(t, 0.0) @ w2_ref[...] + b2_ref[...]
        if relu_out:
            o = jnp.maximum(o, 0.0)
        o_ref[...] = o

    nspec = pl.BlockSpec((_NB, hd), lambda i: (i, 0))
    sspec = pl.BlockSpec((_NC, _NB, hd), lambda i: (0, i, 0))
    cspec = pl.BlockSpec((_NC, _NB, hd), lambda i: (0, i, 0))
    ws = [w1x, w1a, b1, w2, b2]
    return pl.pallas_call(
        body,
        grid=(grid,),
        in_specs=[nspec, sspec, cspec] + [_wspec(w) for w in ws],
        out_specs=nspec,
        out_shape=jax.ShapeDtypeStruct((n, hd), jnp.float32),
    )(x_in, sums, cnt, *ws)


def kernel(x, edge_index, edge_attr,
           e1_w1, e1_b1, e1_w2, e1_b2,
           n1a_w1, n1a_b1, n1a_w2, n1a_b2,
           n1b_w1, n1b_b1, n1b_w2, n1b_b2,
           e2_w1, e2_b1, e2_w2, e2_b2,
           n2a_w1, n2a_b1, n2a_w2, n2a_b2,
           n2b_w1, n2b_b1, n2b_w2, n2b_b2):
    n, d = x.shape
    h = e1_w2.shape[0]
    row = edge_index[0]
    col = edge_index[1]

    g1r, g1c = _gather_two(x, row, col)
    cnt = _scatter_ones(row, n, h, x.dtype, g1r)
    ea, hmsg = _edge1(
        g1r, g1c, edge_attr,
        e1_w1[:d], e1_w1[d:2 * d], e1_w1[2 * d:], e1_b1[None, :],
        e1_w2, e1_b2[None, :],
        n1a_w1[:d], n1a_w1[d:], n1a_b1[None, :], n1a_w2, n1a_b2[None, :])
    sums1 = _scatter_sum(hmsg, row, n, cnt)
    x1 = _node(x, sums1, cnt,
               n1b_w1[:d], n1b_w1[d:], n1b_b1[None, :],
               n1b_w2, n1b_b2[None, :], relu_out=True)

    g2r, g2c = _gather_two(x1, row, col)
    h2 = _edge2(
        g2r, g2c, ea,
        e2_w1[:h], e2_w1[h:2 * h], e2_w1[2 * h:], e2_b1[None, :],
        e2_w2, e2_b2[None, :],
        n2a_w1[:h], n2a_w1[h:], n2a_b1[None, :], n2a_w2, n2a_b2[None, :])
    sums2 = _scatter_sum(h2, row, n, g2c)
    w2p = jnp.pad(n2b_w2, ((0, 0), (0, h - n2b_w2.shape[1])))
    b2p = jnp.pad(n2b_b2, (0, h - n2b_b2.shape[0]))
    outp = _node(x1, sums2, cnt,
                 n2b_w1[:h], n2b_w1[h:], n2b_b1[None, :],
                 w2p, b2p[None, :], relu_out=False)
    return outp[:, :n2b_w2.shape[1]]


# speedup vs baseline: 2.9802x; 1.1971x over previous
import functools

import jax
import jax.numpy as jnp
from jax import lax
from jax.experimental import pallas as pl
from jax.experimental.pallas import tpu as pltpu
from jax.experimental.pallas import tpu_sc as plsc

_NC = 2
_NS = 16
_NW = _NC * _NS
_CHUNK = 80


def _sc_mesh():
    return plsc.VectorSubcoreMesh(
        core_axis_name="c", subcore_axis_name="s",
        num_cores=_NC, num_subcores=_NS)


def _gather_two(table, row, col):
    n, hd = table.shape
    e = row.shape[0]
    ept = e // _NW
    nch = ept // _CHUNK
    assert ept * _NW == e and nch * _CHUNK == ept and nch >= 4

    scratch = ([pltpu.VMEM((_CHUNK,), jnp.int32)] * 4
               + [pltpu.VMEM((_CHUNK, hd), table.dtype)] * 4
               + [pltpu.SemaphoreType.DMA] * 12)

    @functools.partial(
        pl.kernel,
        out_type=[jax.ShapeDtypeStruct((e, hd), table.dtype),
                  jax.ShapeDtypeStruct((e, hd), table.dtype)],
        mesh=_sc_mesh(),
        scratch_types=scratch,
    )
    def gk(tab_hbm, row_hbm, col_hbm, outr_hbm, outc_hbm,
           ir0, ic0, ir1, ic1, br0, bc0, br1, bc1, *sems):
        idx = ((ir0, ic0), (ir1, ic1))
        buf = ((br0, bc0), (br1, bc1))
        sa = (sems[0:2], sems[2:4])
        sb = (sems[4:6], sems[6:8])
        sc = (sems[8:10], sems[10:12])
        wid = lax.axis_index("s") * _NC + lax.axis_index("c")
        base = wid * ept

        def a_start(g, s):
            off = base + g * _CHUNK
            pltpu.async_copy(row_hbm.at[pl.ds(off, _CHUNK)], idx[s][0], sa[s][0])
            pltpu.async_copy(col_hbm.at[pl.ds(off, _CHUNK)], idx[s][1], sa[s][1])

        def a_wait(g, s):
            off = base + g * _CHUNK
            pltpu.make_async_copy(row_hbm.at[pl.ds(off, _CHUNK)], idx[s][0], sa[s][0]).wait()
            pltpu.make_async_copy(col_hbm.at[pl.ds(off, _CHUNK)], idx[s][1], sa[s][1]).wait()

        def b_start(s):
            pltpu.async_copy(tab_hbm.at[idx[s][0]], buf[s][0], sb[s][0])
            pltpu.async_copy(tab_hbm.at[idx[s][1]], buf[s][1], sb[s][1])

        def b_wait(s):
            pltpu.make_async_copy(tab_hbm.at[idx[s][0]], buf[s][0], sb[s][0]).wait()
            pltpu.make_async_copy(tab_hbm.at[idx[s][1]], buf[s][1], sb[s][1]).wait()

        def c_start(g, s):
            off = base + g * _CHUNK
            pltpu.async_copy(buf[s][0], outr_hbm.at[pl.ds(off, _CHUNK), :], sc[s][0])
            pltpu.async_copy(buf[s][1], outc_hbm.at[pl.ds(off, _CHUNK), :], sc[s][1])

        def c_wait(g, s):
            off = base + g * _CHUNK
            pltpu.make_async_copy(buf[s][0], outr_hbm.at[pl.ds(off, _CHUNK), :], sc[s][0]).wait()
            pltpu.make_async_copy(buf[s][1], outc_hbm.at[pl.ds(off, _CHUNK), :], sc[s][1]).wait()

        a_start(0, 0)
        a_start(1, 1)
        a_wait(0, 0)
        b_start(0)

        def pair(k, carry):
            for s in range(2):
                g = 2 * k + s
                o = 1 - s

                @pl.when(g < nch)
                def _(g=g, s=s, o=o):
                    b_wait(s)
                    c_start(g, s)

                    @pl.when(g + 2 < nch)
                    def _():
                        a_start(g + 2, s)

                    @pl.when(g + 1 < nch)
                    def _():
                        @pl.when(g >= 1)
                        def _():
                            c_wait(g - 1, o)

                        a_wait(g + 1, o)
                        b_start(o)
            return carry

        lax.fori_loop(0, (nch + 1) // 2, pair, 0)
        c_wait(nch - 2, (nch - 2) % 2)
        c_wait(nch - 1, (nch - 1) % 2)

    return gk(table, row, col)


def _acc_pad(n):
    npt = -(-n // (_NS * _CHUNK)) * _CHUNK
    return npt, npt * _NS


def _scatter_sum(vals, row, n, token):
    e, hd = vals.shape
    ept = e // _NW
    nch = ept // _CHUNK
    npt, np_ = _acc_pad(n)
    assert ept * _NW == e and nch * _CHUNK == ept

    zeros_h = jnp.zeros((np_, hd), vals.dtype)

    @functools.partial(
        pl.kernel,
        out_type=[jax.ShapeDtypeStruct((_NC, np_, hd), vals.dtype)],
        mesh=_sc_mesh(),
        scratch_types=[pltpu.VMEM((_CHUNK,), jnp.int32),
                       pltpu.VMEM((_CHUNK, hd), vals.dtype),
                       pltpu.VMEM_SHARED((np_, hd), vals.dtype)],
    )
    def sk(vals_hbm, row_hbm, zh_hbm, tok_hbm, sums_hbm, idx_v, vals_v,
           acc_sh):
        cid = lax.axis_index("c")
        sid = lax.axis_index("s")
        base = (sid * _NC + cid) * ept
        nbase = sid * npt
        nzch = npt // _CHUNK
        pltpu.sync_copy(zh_hbm.at[pl.ds(0, _CHUNK), :], vals_v)

        def zbody(i, carry):
            pltpu.sync_copy(vals_v, acc_sh.at[pl.ds(nbase + i * _CHUNK, _CHUNK), :])
            return carry

        lax.fori_loop(0, nzch, zbody, 0)
        plsc.subcore_barrier()

        def body(i, carry):
            off = base + i * _CHUNK
            pltpu.sync_copy(row_hbm.at[pl.ds(off, _CHUNK)], idx_v)
            pltpu.sync_copy(vals_hbm.at[pl.ds(off, _CHUNK), :], vals_v)
            pltpu.sync_copy(vals_v, acc_sh.at[idx_v], add=True)
            return carry

        lax.fori_loop(0, nch, body, 0)
        plsc.subcore_barrier()

        def wbody(i, carry):
            zoff = nbase + i * _CHUNK
            pltpu.sync_copy(acc_sh.at[pl.ds(zoff, _CHUNK), :], vals_v)
            pltpu.sync_copy(vals_v, sums_hbm.at[cid, pl.ds(zoff, _CHUNK), :])
            return carry

        lax.fori_loop(0, nzch, wbody, 0)

    return sk(vals, row, zeros_h, token)[0]


def _scatter_ones(row, n, hd, dtype, token):
    e = row.shape[0]
    ept = e // _NW
    nch = ept // _CHUNK
    npt, np_ = _acc_pad(n)

    zeros_h = jnp.zeros((np_, hd), dtype)
    ones = jnp.ones((_CHUNK, hd), dtype)

    @functools.partial(
        pl.kernel,
        out_type=[jax.ShapeDtypeStruct((_NC, np_, hd), dtype)],
        mesh=_sc_mesh(),
        scratch_types=[pltpu.VMEM((_CHUNK,), jnp.int32),
                       pltpu.VMEM((_CHUNK, hd), dtype),
                       pltpu.VMEM((_CHUNK, hd), dtype),
                       pltpu.VMEM_SHARED((np_, hd), dtype)],
    )
    def ck(row_hbm, zh_hbm, ones_hbm, tok_hbm, cnt_hbm, idx_v, ones_v,
           stage_v, acc_sh):
        cid = lax.axis_index("c")
        sid = lax.axis_index("s")
        base = (sid * _NC + cid) * ept
        nbase = sid * npt
        nzch = npt // _CHUNK
        pltpu.sync_copy(zh_hbm.at[pl.ds(0, _CHUNK), :], stage_v)

        def zbody(i, carry):
            pltpu.sync_copy(stage_v, acc_sh.at[pl.ds(nbase + i * _CHUNK, _CHUNK), :])
            return carry

        lax.fori_loop(0, nzch, zbody, 0)
        pltpu.sync_copy(ones_hbm, ones_v)
        plsc.subcore_barrier()

        def body(i, carry):
            off = base + i * _CHUNK
            pltpu.sync_copy(row_hbm.at[pl.ds(off, _CHUNK)], idx_v)
            pltpu.sync_copy(ones_v, acc_sh.at[idx_v], add=True)
            return carry

        lax.fori_loop(0, nch, body, 0)
        plsc.subcore_barrier()

        def wbody(i, carry):
            zoff = nbase + i * _CHUNK
            pltpu.sync_copy(acc_sh.at[pl.ds(zoff, _CHUNK), :], stage_v)
            pltpu.sync_copy(stage_v, cnt_hbm.at[cid, pl.ds(zoff, _CHUNK), :])
            return carry

        lax.fori_loop(0, nzch, wbody, 0)

    return ck(row, zeros_h, ones, token)[0]


def _wspec(w):
    nd = w.ndim
    return pl.BlockSpec(w.shape, lambda i, _nd=nd: (0,) * _nd)


_EB = 3200


def _edge1(gr, gc, eattr, w1a, w1b, w1c, b1, w2, b2,
           nw1x, nw1e, nb1, nw2, nb2):
    e, hd = gr.shape
    de = eattr.shape[1]
    grid = e // _EB
    assert grid * _EB == e

    def body(gr_ref, gc_ref, at_ref, w1a_ref, w1b_ref, w1c_ref, b1_ref,
             w2_ref, b2_ref, nw1x_ref, nw1e_ref, nb1_ref, nw2_ref, nb2_ref,
             ea_ref, h_ref):
        grb = gr_ref[...]
        gcb = gc_ref[...]
        t = (grb @ w1a_ref[...] + gcb @ w1b_ref[...]
             + at_ref[...] @ w1c_ref[...] + b1_ref[...])
        ea = jnp.maximum(t, 0.0) @ w2_ref[...] + b2_ref[...]
        ea_ref[...] = ea
        u = gcb @ nw1x_ref[...] + ea @ nw1e_ref[...] + nb1_ref[...]
        h_ref[...] = jnp.maximum(u, 0.0) @ nw2_ref[...] + nb2_ref[...]

    espec = pl.BlockSpec((_EB, hd), lambda i: (i, 0))
    aspec = pl.BlockSpec((_EB, de), lambda i: (i, 0))
    ws = [w1a, w1b, w1c, b1, w2, b2, nw1x, nw1e, nb1, nw2, nb2]
    return pl.pallas_call(
        body,
        grid=(grid,),
        in_specs=[espec, espec, aspec] + [_wspec(w) for w in ws],
        out_specs=[espec, espec],
        out_shape=[jax.ShapeDtypeStruct((e, hd), jnp.float32)] * 2,
    )(gr, gc, eattr, *ws)


def _edge2(gr, gc, ea, w1a, w1b, w1c, b1, w2, b2,
           nw1x, nw1e, nb1, nw2, nb2):
    e, hd = gr.shape
    grid = e // _EB
    assert grid * _EB == e

    def body(gr_ref, gc_ref, ea_ref, w1a_ref, w1b_ref, w1c_ref, b1_ref,
             w2_ref, b2_ref, nw1x_ref, nw1e_ref, nb1_ref, nw2_ref, nb2_ref,
             h_ref):
        grb = gr_ref[...]
        gcb = gc_ref[...]
        t = (grb @ w1a_ref[...] + gcb @ w1b_ref[...]
             + ea_ref[...] @ w1c_ref[...] + b1_ref[...])
        ea2 = jnp.maximum(t, 0.0) @ w2_ref[...] + b2_ref[...]
        u = gcb @ nw1x_ref[...] + ea2 @ nw1e_ref[...] + nb1_ref[...]
        h_ref[...] = jnp.maximum(u, 0.0) @ nw2_ref[...] + nb2_ref[...]

    espec = pl.BlockSpec((_EB, hd), lambda i: (i, 0))
    ws = [w1a, w1b, w1c, b1, w2, b2, nw1x, nw1e, nb1, nw2, nb2]
    return pl.pallas_call(
        body,
        grid=(grid,),
        in_specs=[espec, espec, espec] + [_wspec(w) for w in ws],
        out_specs=espec,
        out_shape=jax.ShapeDtypeStruct((e, hd), jnp.float32),
    )(gr, gc, ea, *ws)


_NB = 2000


def _node(x_in, sums, cnt, w1x, w1a, b1, w2, b2, relu_out):
    n, hd = x_in.shape
    grid = n // _NB
    assert grid * _NB == n

    def body(x_ref, s_ref, c_ref, w1x_ref, w1a_ref, b1_ref, w2_ref, b2_ref,
             o_ref):
        s = s_ref[0] + s_ref[1]
        c = c_ref[0] + c_ref[1]
        agg = s / jnp.maximum(c[:, 0:1], 1.0)
        t = x_ref[...] @ w1x_ref[...] + agg @ w1a_ref[...] + b1_ref[...]
        o = jnp.maximum(t, 0.0) @ w2_ref[...] + b2_ref[...]
        if relu_out:
            o = jnp.maximum(o, 0.0)
        o_ref[...] = o

    nspec = pl.BlockSpec((_NB, hd), lambda i: (i, 0))
    sspec = pl.BlockSpec((_NC, _NB, hd), lambda i: (0, i, 0))
    cspec = pl.BlockSpec((_NC, _NB, hd), lambda i: (0, i, 0))
    ws = [w1x, w1a, b1, w2, b2]
    return pl.pallas_call(
        body,
        grid=(grid,),
        in_specs=[nspec, sspec, cspec] + [_wspec(w) for w in ws],
        out_specs=nspec,
        out_shape=jax.ShapeDtypeStruct((n, hd), jnp.float32),
    )(x_in, sums, cnt, *ws)


def kernel(x, edge_index, edge_attr,
           e1_w1, e1_b1, e1_w2, e1_b2,
           n1a_w1, n1a_b1, n1a_w2, n1a_b2,
           n1b_w1, n1b_b1, n1b_w2, n1b_b2,
           e2_w1, e2_b1, e2_w2, e2_b2,
           n2a_w1, n2a_b1, n2a_w2, n2a_b2,
           n2b_w1, n2b_b1, n2b_w2, n2b_b2):
    n, d = x.shape
    h = e1_w2.shape[0]
    row = edge_index[0]
    col = edge_index[1]

    g1r, g1c = _gather_two(x, row, col)
    cnt = _scatter_ones(row, n, h, x.dtype, g1r)
    ea, hmsg = _edge1(
        g1r, g1c, edge_attr,
        e1_w1[:d], e1_w1[d:2 * d], e1_w1[2 * d:], e1_b1[None, :],
        e1_w2, e1_b2[None, :],
        n1a_w1[:d], n1a_w1[d:], n1a_b1[None, :], n1a_w2, n1a_b2[None, :])
    sums1 = _scatter_sum(hmsg, row, n, cnt)
    x1 = _node(x, sums1, cnt,
               n1b_w1[:d], n1b_w1[d:], n1b_b1[None, :],
               n1b_w2, n1b_b2[None, :], relu_out=True)

    g2r, g2c = _gather_two(x1, row, col)
    h2 = _edge2(
        g2r, g2c, ea,
        e2_w1[:h], e2_w1[h:2 * h], e2_w1[2 * h:], e2_b1[None, :],
        e2_w2, e2_b2[None, :],
        n2a_w1[:h], n2a_w1[h:], n2a_b1[None, :], n2a_w2, n2a_b2[None, :])
    sums2 = _scatter_sum(h2, row, n, g2c)
    w2p = jnp.pad(n2b_w2, ((0, 0), (0, h - n2b_w2.shape[1])))
    b2p = jnp.pad(n2b_b2, (0, h - n2b_b2.shape[0]))
    outp = _node(x1, sums2, cnt,
                 n2b_w1[:h], n2b_w1[h:], n2b_b1[None, :],
                 w2p, b2p[None, :], relu_out=False)
    return outp[:, :n2b_w2.shape[1]]


# speedup vs baseline: 3.5430x; 1.1889x over previous
import functools

import jax
import jax.numpy as jnp
from jax import lax
from jax.experimental import pallas as pl
from jax.experimental.pallas import tpu as pltpu
from jax.experimental.pallas import tpu_sc as plsc

_NC = 2
_NS = 16
_NW = _NC * _NS
_CHUNK = 80


def _sc_mesh():
    return plsc.VectorSubcoreMesh(
        core_axis_name="c", subcore_axis_name="s",
        num_cores=_NC, num_subcores=_NS)


def _gather_two(table, row, col):
    n, hd = table.shape
    e = row.shape[0]
    ept = e // _NW
    nch = ept // _CHUNK
    assert ept * _NW == e and nch * _CHUNK == ept and nch >= 4

    scratch = ([pltpu.VMEM((_CHUNK,), jnp.int32)] * 4
               + [pltpu.VMEM((_CHUNK, hd), table.dtype)] * 4
               + [pltpu.SemaphoreType.DMA] * 12)

    @functools.partial(
        pl.kernel,
        out_type=[jax.ShapeDtypeStruct((e, hd), table.dtype),
                  jax.ShapeDtypeStruct((e, hd), table.dtype)],
        mesh=_sc_mesh(),
        scratch_types=scratch,
    )
    def gk(tab_hbm, row_hbm, col_hbm, outr_hbm, outc_hbm,
           ir0, ic0, ir1, ic1, br0, bc0, br1, bc1, *sems):
        idx = ((ir0, ic0), (ir1, ic1))
        buf = ((br0, bc0), (br1, bc1))
        sa = (sems[0:2], sems[2:4])
        sb = (sems[4:6], sems[6:8])
        sc = (sems[8:10], sems[10:12])
        wid = lax.axis_index("s") * _NC + lax.axis_index("c")
        base = wid * ept

        def a_start(g, s):
            off = base + g * _CHUNK
            pltpu.async_copy(row_hbm.at[pl.ds(off, _CHUNK)], idx[s][0], sa[s][0])
            pltpu.async_copy(col_hbm.at[pl.ds(off, _CHUNK)], idx[s][1], sa[s][1])

        def a_wait(g, s):
            off = base + g * _CHUNK
            pltpu.make_async_copy(row_hbm.at[pl.ds(off, _CHUNK)], idx[s][0], sa[s][0]).wait()
            pltpu.make_async_copy(col_hbm.at[pl.ds(off, _CHUNK)], idx[s][1], sa[s][1]).wait()

        def b_start(s):
            pltpu.async_copy(tab_hbm.at[idx[s][0]], buf[s][0], sb[s][0])
            pltpu.async_copy(tab_hbm.at[idx[s][1]], buf[s][1], sb[s][1])

        def b_wait(s):
            pltpu.make_async_copy(tab_hbm.at[idx[s][0]], buf[s][0], sb[s][0]).wait()
            pltpu.make_async_copy(tab_hbm.at[idx[s][1]], buf[s][1], sb[s][1]).wait()

        def c_start(g, s):
            off = base + g * _CHUNK
            pltpu.async_copy(buf[s][0], outr_hbm.at[pl.ds(off, _CHUNK), :], sc[s][0])
            pltpu.async_copy(buf[s][1], outc_hbm.at[pl.ds(off, _CHUNK), :], sc[s][1])

        def c_wait(g, s):
            off = base + g * _CHUNK
            pltpu.make_async_copy(buf[s][0], outr_hbm.at[pl.ds(off, _CHUNK), :], sc[s][0]).wait()
            pltpu.make_async_copy(buf[s][1], outc_hbm.at[pl.ds(off, _CHUNK), :], sc[s][1]).wait()

        a_start(0, 0)
        a_start(1, 1)
        a_wait(0, 0)
        b_start(0)

        def pair(k, carry):
            for s in range(2):
                g = 2 * k + s
                o = 1 - s

                @pl.when(g < nch)
                def _(g=g, s=s, o=o):
                    b_wait(s)
                    c_start(g, s)

                    @pl.when(g + 2 < nch)
                    def _():
                        a_start(g + 2, s)

                    @pl.when(g + 1 < nch)
                    def _():
                        @pl.when(g >= 1)
                        def _():
                            c_wait(g - 1, o)

                        a_wait(g + 1, o)
                        b_start(o)
            return carry

        lax.fori_loop(0, (nch + 1) // 2, pair, 0)
        c_wait(nch - 2, (nch - 2) % 2)
        c_wait(nch - 1, (nch - 1) % 2)

    return gk(table, row, col)


def _acc_pad(n):
    npt = -(-n // (_NS * _CHUNK)) * _CHUNK
    return npt, npt * _NS


def _scatter_sum(vals, row, n, token):
    e, hd = vals.shape
    ept = e // _NW
    nch = ept // _CHUNK
    npt, np_ = _acc_pad(n)
    assert ept * _NW == e and nch * _CHUNK == ept

    zeros_h = jnp.zeros((np_, hd), vals.dtype)

    @functools.partial(
        pl.kernel,
        out_type=[jax.ShapeDtypeStruct((_NC, np_, hd), vals.dtype)],
        mesh=_sc_mesh(),
        scratch_types=[pltpu.VMEM((_CHUNK,), jnp.int32),
                       pltpu.VMEM((_CHUNK,), jnp.int32),
                       pltpu.VMEM((_CHUNK, hd), vals.dtype),
                       pltpu.VMEM((_CHUNK, hd), vals.dtype),
                       pltpu.SemaphoreType.DMA,
                       pltpu.SemaphoreType.DMA,
                       pltpu.SemaphoreType.DMA,
                       pltpu.SemaphoreType.DMA,
                       pltpu.VMEM_SHARED((np_, hd), vals.dtype)],
    )
    def sk(vals_hbm, row_hbm, zh_hbm, tok_hbm, sums_hbm, i0, i1, v0, v1,
           si0, si1, sv0, sv1, acc_sh):
        idx = (i0, i1)
        vv = (v0, v1)
        sem = ((si0, sv0), (si1, sv1))
        cid = lax.axis_index("c")
        sid = lax.axis_index("s")
        base = (sid * _NC + cid) * ept
        nbase = sid * npt
        nzch = npt // _CHUNK
        pltpu.sync_copy(zh_hbm.at[pl.ds(0, _CHUNK), :], v0)

        def zbody(i, carry):
            pltpu.sync_copy(v0, acc_sh.at[pl.ds(nbase + i * _CHUNK, _CHUNK), :])
            return carry

        lax.fori_loop(0, nzch, zbody, 0)
        plsc.subcore_barrier()

        def ld_start(g, s):
            off = base + g * _CHUNK
            pltpu.async_copy(row_hbm.at[pl.ds(off, _CHUNK)], idx[s], sem[s][0])
            pltpu.async_copy(vals_hbm.at[pl.ds(off, _CHUNK), :], vv[s], sem[s][1])

        def ld_wait(g, s):
            off = base + g * _CHUNK
            pltpu.make_async_copy(row_hbm.at[pl.ds(off, _CHUNK)], idx[s], sem[s][0]).wait()
            pltpu.make_async_copy(vals_hbm.at[pl.ds(off, _CHUNK), :], vv[s], sem[s][1]).wait()

        ld_start(0, 0)
        ld_start(1, 1)

        def pair(k, carry):
            for s in range(2):
                g = 2 * k + s

                @pl.when(g < nch)
                def _(g=g, s=s):
                    ld_wait(g, s)
                    pltpu.sync_copy(vv[s], acc_sh.at[idx[s]], add=True)

                    @pl.when(g + 2 < nch)
                    def _():
                        ld_start(g + 2, s)
            return carry

        lax.fori_loop(0, (nch + 1) // 2, pair, 0)
        plsc.subcore_barrier()

        def wbody(i, carry):
            zoff = nbase + i * _CHUNK
            pltpu.sync_copy(acc_sh.at[pl.ds(zoff, _CHUNK), :], v0)
            pltpu.sync_copy(v0, sums_hbm.at[cid, pl.ds(zoff, _CHUNK), :])
            return carry

        lax.fori_loop(0, nzch, wbody, 0)

    return sk(vals, row, zeros_h, token)[0]


def _scatter_ones(row, n, hd, dtype, token):
    e = row.shape[0]
    ept = e // _NW
    nch = ept // _CHUNK
    npt, np_ = _acc_pad(n)

    zeros_h = jnp.zeros((np_, hd), dtype)
    ones = jnp.ones((_CHUNK, hd), dtype)

    @functools.partial(
        pl.kernel,
        out_type=[jax.ShapeDtypeStruct((_NC, np_, hd), dtype)],
        mesh=_sc_mesh(),
        scratch_types=[pltpu.VMEM((_CHUNK,), jnp.int32),
                       pltpu.VMEM((_CHUNK,), jnp.int32),
                       pltpu.VMEM((_CHUNK, hd), dtype),
                       pltpu.VMEM((_CHUNK, hd), dtype),
                       pltpu.SemaphoreType.DMA,
                       pltpu.SemaphoreType.DMA,
                       pltpu.VMEM_SHARED((np_, hd), dtype)],
    )
    def ck(row_hbm, zh_hbm, ones_hbm, tok_hbm, cnt_hbm, i0, i1, ones_v,
           stage_v, s0, s1, acc_sh):
        idx = (i0, i1)
        sem = (s0, s1)
        cid = lax.axis_index("c")
        sid = lax.axis_index("s")
        base = (sid * _NC + cid) * ept
        nbase = sid * npt
        nzch = npt // _CHUNK
        pltpu.sync_copy(zh_hbm.at[pl.ds(0, _CHUNK), :], stage_v)

        def zbody(i, carry):
            pltpu.sync_copy(stage_v, acc_sh.at[pl.ds(nbase + i * _CHUNK, _CHUNK), :])
            return carry

        lax.fori_loop(0, nzch, zbody, 0)
        pltpu.sync_copy(ones_hbm, ones_v)
        plsc.subcore_barrier()

        def ld_start(g, s):
            off = base + g * _CHUNK
            pltpu.async_copy(row_hbm.at[pl.ds(off, _CHUNK)], idx[s], sem[s])

        def ld_wait(g, s):
            off = base + g * _CHUNK
            pltpu.make_async_copy(row_hbm.at[pl.ds(off, _CHUNK)], idx[s], sem[s]).wait()

        ld_start(0, 0)
        ld_start(1, 1)

        def pair(k, carry):
            for s in range(2):
                g = 2 * k + s

                @pl.when(g < nch)
                def _(g=g, s=s):
                    ld_wait(g, s)
                    pltpu.sync_copy(ones_v, acc_sh.at[idx[s]], add=True)

                    @pl.when(g + 2 < nch)
                    def _():
                        ld_start(g + 2, s)
            return carry

        lax.fori_loop(0, (nch + 1) // 2, pair, 0)
        plsc.subcore_barrier()

        def wbody(i, carry):
            zoff = nbase + i * _CHUNK
            pltpu.sync_copy(acc_sh.at[pl.ds(zoff, _CHUNK), :], stage_v)
            pltpu.sync_copy(stage_v, cnt_hbm.at[cid, pl.ds(zoff, _CHUNK), :])
            return carry

        lax.fori_loop(0, nzch, wbody, 0)

    return ck(row, zeros_h, ones, token)[0]


def _wspec(w):
    nd = w.ndim
    return pl.BlockSpec(w.shape, lambda i, _nd=nd: (0,) * _nd)


_EB = 3200


def _edge1(gr, gc, eattr, w1a, w1b, w1c, b1, w2, b2,
           nw1x, nw1e, nb1, nw2, nb2):
    e, hd = gr.shape
    de = eattr.shape[1]
    grid = e // _EB
    assert grid * _EB == e

    def body(gr_ref, gc_ref, at_ref, w1a_ref, w1b_ref, w1c_ref, b1_ref,
             w2_ref, b2_ref, nw1x_ref, nw1e_ref, nb1_ref, nw2_ref, nb2_ref,
             ea_ref, h_ref):
        grb = gr_ref[...]
        gcb = gc_ref[...]
        t = (grb @ w1a_ref[...] + gcb @ w1b_ref[...]
             + at_ref[...] @ w1c_ref[...] + b1_ref[...])
        ea = jnp.maximum(t, 0.0) @ w2_ref[...] + b2_ref[...]
        ea_ref[...] = ea
        u = gcb @ nw1x_ref[...] + ea @ nw1e_ref[...] + nb1_ref[...]
        h_ref[...] = jnp.maximum(u, 0.0) @ nw2_ref[...] + nb2_ref[...]

    espec = pl.BlockSpec((_EB, hd), lambda i: (i, 0))
    aspec = pl.BlockSpec((_EB, de), lambda i: (i, 0))
    ws = [w1a, w1b, w1c, b1, w2, b2, nw1x, nw1e, nb1, nw2, nb2]
    return pl.pallas_call(
        body,
        grid=(grid,),
        in_specs=[espec, espec, aspec] + [_wspec(w) for w in ws],
        out_specs=[espec, espec],
        out_shape=[jax.ShapeDtypeStruct((e, hd), jnp.float32)] * 2,
    )(gr, gc, eattr, *ws)


def _edge2(gr, gc, ea, w1a, w1b, w1c, b1, w2, b2,
           nw1x, nw1e, nb1, nw2, nb2):
    e, hd = gr.shape
    grid = e // _EB
    assert grid * _EB == e

    def body(gr_ref, gc_ref, ea_ref, w1a_ref, w1b_ref, w1c_ref, b1_ref,
             w2_ref, b2_ref, nw1x_ref, nw1e_ref, nb1_ref, nw2_ref, nb2_ref,
             h_ref):
        grb = gr_ref[...]
        gcb = gc_ref[...]
        t = (grb @ w1a_ref[...] + gcb @ w1b_ref[...]
             + ea_ref[...] @ w1c_ref[...] + b1_ref[...])
        ea2 = jnp.maximum(t, 0.0) @ w2_ref[...] + b2_ref[...]
        u = gcb @ nw1x_ref[...] + ea2 @ nw1e_ref[...] + nb1_ref[...]
        h_ref[...] = jnp.maximum(u, 0.0) @ nw2_ref[...] + nb2_ref[...]

    espec = pl.BlockSpec((_EB, hd), lambda i: (i, 0))
    ws = [w1a, w1b, w1c, b1, w2, b2, nw1x, nw1e, nb1, nw2, nb2]
    return pl.pallas_call(
        body,
        grid=(grid,),
        in_specs=[espec, espec, espec] + [_wspec(w) for w in ws],
        out_specs=espec,
        out_shape=jax.ShapeDtypeStruct((e, hd), jnp.float32),
    )(gr, gc, ea, *ws)


_NB = 2000


def _node(x_in, sums, cnt, w1x, w1a, b1, w2, b2, relu_out):
    n, hd = x_in.shape
    grid = n // _NB
    assert grid * _NB == n

    def body(x_ref, s_ref, c_ref, w1x_ref, w1a_ref, b1_ref, w2_ref, b2_ref,
             o_ref):
        s = s_ref[0] + s_ref[1]
        c = c_ref[0] + c_ref[1]
        agg = s / jnp.maximum(c[:, 0:1], 1.0)
        t = x_ref[...] @ w1x_ref[...] + agg @ w1a_ref[...] + b1_ref[...]
        o = jnp.maximum(t, 0.0) @ w2_ref[...] + b2_ref[...]
        if relu_out:
            o = jnp.maximum(o, 0.0)
        o_ref[...] = o

    nspec = pl.BlockSpec((_NB, hd), lambda i: (i, 0))
    sspec = pl.BlockSpec((_NC, _NB, hd), lambda i: (0, i, 0))
    cspec = pl.BlockSpec((_NC, _NB, hd), lambda i: (0, i, 0))
    ws = [w1x, w1a, b1, w2, b2]
    return pl.pallas_call(
        body,
        grid=(grid,),
        in_specs=[nspec, sspec, cspec] + [_wspec(w) for w in ws],
        out_specs=nspec,
        out_shape=jax.ShapeDtypeStruct((n, hd), jnp.float32),
    )(x_in, sums, cnt, *ws)


def kernel(x, edge_index, edge_attr,
           e1_w1, e1_b1, e1_w2, e1_b2,
           n1a_w1, n1a_b1, n1a_w2, n1a_b2,
           n1b_w1, n1b_b1, n1b_w2, n1b_b2,
           e2_w1, e2_b1, e2_w2, e2_b2,
           n2a_w1, n2a_b1, n2a_w2, n2a_b2,
           n2b_w1, n2b_b1, n2b_w2, n2b_b2):
    n, d = x.shape
    h = e1_w2.shape[0]
    row = edge_index[0]
    col = edge_index[1]

    g1r, g1c = _gather_two(x, row, col)
    cnt = _scatter_ones(row, n, h, x.dtype, g1r)
    ea, hmsg = _edge1(
        g1r, g1c, edge_attr,
        e1_w1[:d], e1_w1[d:2 * d], e1_w1[2 * d:], e1_b1[None, :],
        e1_w2, e1_b2[None, :],
        n1a_w1[:d], n1a_w1[d:], n1a_b1[None, :], n1a_w2, n1a_b2[None, :])
    sums1 = _scatter_sum(hmsg, row, n, cnt)
    x1 = _node(x, sums1, cnt,
               n1b_w1[:d], n1b_w1[d:], n1b_b1[None, :],
               n1b_w2, n1b_b2[None, :], relu_out=True)

    g2r, g2c = _gather_two(x1, row, col)
    h2 = _edge2(
        g2r, g2c, ea,
        e2_w1[:h], e2_w1[h:2 * h], e2_w1[2 * h:], e2_b1[None, :],
        e2_w2, e2_b2[None, :],
        n2a_w1[:h], n2a_w1[h:], n2a_b1[None, :], n2a_w2, n2a_b2[None, :])
    sums2 = _scatter_sum(h2, row, n, g2c)
    w2p = jnp.pad(n2b_w2, ((0, 0), (0, h - n2b_w2.shape[1])))
    b2p = jnp.pad(n2b_b2, (0, h - n2b_b2.shape[0]))
    outp = _node(x1, sums2, cnt,
                 n2b_w1[:h], n2b_w1[h:], n2b_b1[None, :],
                 w2p, b2p[None, :], relu_out=False)
    return outp[:, :n2b_w2.shape[1]]
